# Initial kernel scaffold; baseline (speedup 1.0000x reference)
import jax
import jax.numpy as jnp
from jax.experimental import pallas as pl


def kernel(x_FV, adj_FV, x_FP, adj_FP, params):
    raise NotImplementedError("write your pallas kernel here")


# speedup vs baseline: 1.7663x; 1.7663x over previous
import functools

import jax
import jax.numpy as jnp
from jax.experimental import pallas as pl
from jax.experimental.pallas import tpu as pltpu

_N = 512
_NSP = 200
_INTERPRET = False


def _relu(x):
    return jnp.maximum(x, 0.0)


def _dot(a, b):
    return jax.lax.dot_general(a, b, (((1,), (0,)), ((), ())),
                               preferred_element_type=jnp.float32)


def _tdot(a, b):
    return jax.lax.dot_general(a, b, (((0,), (0,)), ((), ())),
                               preferred_element_type=jnp.float32)


def _dot_t(a, b):
    return jax.lax.dot_general(a, b, (((1,), (1,)), ((), ())),
                               preferred_element_type=jnp.float32)


def _eye(n):
    r = jax.lax.broadcasted_iota(jnp.int32, (n, n), 0)
    c = jax.lax.broadcasted_iota(jnp.int32, (n, n), 1)
    return (r == c).astype(jnp.float32)


def _norm_adj(adj, n, add_self_loops):
    A = adj + _eye(n) if add_self_loops else adj
    deg = jnp.sum(A, axis=1, keepdims=True)
    return A / jnp.maximum(deg, 1e-6)


def _net(x, adj, An, w):
    (pW1, pb1, pW2, pb2, pW3, pb3, Wl1, Wl2, Wl3, bl,
     eW1, eb1, eW2, eb2, eW3, eb3) = w
    h1 = _relu(_dot(An, _dot(x, pW1)) + pb1)
    h2 = _relu(_dot(An, _dot(h1, pW2)) + pb2)
    h3 = _relu(_dot(An, _dot(h2, pW3)) + pb3)
    s = _relu(_dot(h1, Wl1) + _dot(h2, Wl2) + _dot(h3, Wl3) + bl)
    m = jnp.max(s, axis=1, keepdims=True)
    e = jnp.exp(s - m)
    s = e / jnp.sum(e, axis=1, keepdims=True)
    z1 = _relu(_dot(An, _dot(x, eW1)) + eb1)
    z2 = _relu(_dot(An, _dot(z1, eW2)) + eb2)
    z3 = _relu(_dot(An, _dot(z2, eW3)) + eb3)
    u1 = _tdot(s, z1)
    u2 = _tdot(s, z2)
    u3 = _tdot(s, z3)
    t = _tdot(s, adj)
    adj_p = _dot(t, s)
    return (u1, u2, u3), adj_p


def _perm_matrix(adj_p):
    n = _NSP
    deg = jnp.sum(adj_p, axis=1, keepdims=True)
    Ap = adj_p / jnp.maximum(deg, 1e-6)
    p = jnp.full((1, n), 1.0 / n, dtype=jnp.float32)
    for _ in range(10):
        p = 0.85 * _dot(p, Ap) + (1.0 - 0.85) / n
    pj = p.reshape(n, 1)
    gt = (pj > p).astype(jnp.float32)
    ioj = jax.lax.broadcasted_iota(jnp.int32, (n, n), 0)
    ioi = jax.lax.broadcasted_iota(jnp.int32, (n, n), 1)
    eq = ((pj == p) & (ioj < ioi)).astype(jnp.float32)
    rank = jnp.sum(gt + eq, axis=0, keepdims=True)
    kk = jax.lax.broadcasted_iota(jnp.int32, (n, n), 0).astype(jnp.float32)
    return (rank == kk).astype(jnp.float32)


def _body(xv_ref, av_ref, xp_ref, ap_ref, *rest):
    out_ref = rest[-1]
    w = [r[...] for r in rest[:-1]]
    wv, wp = tuple(w[0:16]), tuple(w[16:32])
    (aW1a, aW1b, aW1c, aW1d, aW1e, aW1f, ab1, aW2, ab2, aW3, ab3,
     l1a, l1b, l1c, l1bias, l2W, l2bias) = w[32:]

    xv = xv_ref[0]
    av = av_ref[0]
    xp = xp_ref[0]
    ap = ap_ref[0]

    An_v = _norm_adj(av, _N, True)
    An_p = _norm_adj(ap, _N, True)
    (u1, u2, u3), adj1 = _net(xv, av, An_v, wv)
    (v1, v2, v3), adj3 = _net(xp, ap, An_p, wp)

    P1 = _perm_matrix(adj1)
    P3 = _perm_matrix(adj3)
    u1, u2, u3 = _dot(P1, u1), _dot(P1, u2), _dot(P1, u3)
    v1, v2, v3 = _dot(P3, v1), _dot(P3, v2), _dot(P3, v3)
    adj1 = _dot_t(_dot(P1, adj1), P1)
    adj3 = _dot_t(_dot(P3, adj3), P3)

    adjs = adj1 + adj3
    An2 = _norm_adj(adjs, _NSP, True)
    xW1 = (_dot(u1, aW1a) + _dot(u2, aW1b) + _dot(u3, aW1c) +
           _dot(v1, aW1d) + _dot(v2, aW1e) + _dot(v3, aW1f))
    a1 = _relu(_dot(An2, xW1) + ab1)
    a2 = _relu(_dot(An2, _dot(a1, aW2)) + ab2)
    a3 = _relu(_dot(An2, _dot(a2, aW3)) + ab3)

    inv_n = 1.0 / _NSP
    g1 = jnp.sum(a1, axis=0, keepdims=True) * inv_n
    g2 = jnp.sum(a2, axis=0, keepdims=True) * inv_n
    g3 = jnp.sum(a3, axis=0, keepdims=True) * inv_n

    h = _relu(_dot(g1, l1a) + _dot(g2, l1b) + _dot(g3, l1c) + l1bias)
    logits = _dot(h, l2W) + l2bias
    m = jnp.max(logits, axis=1, keepdims=True)
    lse = jnp.log(jnp.sum(jnp.exp(logits - m), axis=1, keepdims=True))
    out_ref[0] = logits - m - lse


def _net_weights(p):
    pool, emb = p['pool'], p['embed']
    Wl = pool['Wl']
    return [pool['W1'], pool['b1'].reshape(1, -1),
            pool['W2'], pool['b2'].reshape(1, -1),
            pool['W3'], pool['b3'].reshape(1, -1),
            Wl[0:100], Wl[100:200], Wl[200:400], pool['bl'].reshape(1, -1),
            emb['W1'], emb['b1'].reshape(1, -1),
            emb['W2'], emb['b2'].reshape(1, -1),
            emb['W3'], emb['b3'].reshape(1, -1)]


def kernel(x_FV, adj_FV, x_FP, adj_FP, params):
    B = x_FV.shape[0]
    af = params['after']
    aW1 = af['W1']
    l1W = params['lin1_W']
    weights = (_net_weights(params['net_FV']) + _net_weights(params['net_FP']) + [
        aW1[0:100], aW1[100:200], aW1[200:300],
        aW1[300:400], aW1[400:500], aW1[500:600],
        af['b1'].reshape(1, -1), af['W2'], af['b2'].reshape(1, -1),
        af['W3'], af['b3'].reshape(1, -1),
        l1W[0:400], l1W[400:800], l1W[800:1200],
        params['lin1_b'].reshape(1, -1),
        params['lin2_W'], params['lin2_b'].reshape(1, -1)])

    data_specs = [
        pl.BlockSpec((1, _N, x_FV.shape[2]), lambda b: (b, 0, 0)),
        pl.BlockSpec((1, _N, _N), lambda b: (b, 0, 0)),
        pl.BlockSpec((1, _N, x_FV.shape[2]), lambda b: (b, 0, 0)),
        pl.BlockSpec((1, _N, _N), lambda b: (b, 0, 0)),
    ]
    w_specs = [pl.BlockSpec(w.shape, functools.partial(
        lambda nd, b: (0,) * nd, w.ndim)) for w in weights]

    out = pl.pallas_call(
        _body,
        grid=(B,),
        in_specs=data_specs + w_specs,
        out_specs=pl.BlockSpec((1, 1, 585), lambda b: (b, 0, 0)),
        out_shape=jax.ShapeDtypeStruct((B, 1, 585), jnp.float32),
        compiler_params=pltpu.CompilerParams(
            dimension_semantics=("parallel",)),
        interpret=_INTERPRET,
    )(x_FV, adj_FV, x_FP, adj_FP, *weights)
    return out.reshape(B, 585)
